# baseline (speedup 1.0000x reference)
import functools

import jax
import jax.numpy as jnp
from jax import lax
from jax.experimental import pallas as pl
from jax.experimental.pallas import tpu as pltpu
from jax.experimental.pallas import tpu_sc as plsc

VOCAB = 1000000
EMB_DIM = 64
NUM_CLASS = 7
BATCH = 16384

NC = 2
NS = 16
L = 16
NW = NC * NS
BPW = BATCH // NW
IDX_CHUNK = 128
N_IDX_CHUNKS = BPW // IDX_CHUNK
ROWS_PER_BLK = 4 * L
N_BLKS = BPW // ROWS_PER_BLK


def _sc_call(node, table, w_splat, b_splat):
    mesh = plsc.VectorSubcoreMesh(core_axis_name="c", subcore_axis_name="s")

    @functools.partial(
        pl.kernel,
        mesh=mesh,
        compiler_params=pltpu.CompilerParams(
            needs_layout_passes=False, use_tc_tiling_on_sc=False
        ),
        out_type=jax.ShapeDtypeStruct((BATCH, NUM_CLASS), jnp.float32),
        scratch_types=[
            pltpu.VMEM((N_IDX_CHUNKS, IDX_CHUNK), jnp.int32),
            pltpu.VMEM((BPW, EMB_DIM), jnp.float32),
            pltpu.VMEM((NUM_CLASS * EMB_DIM, L), jnp.float32),
            pltpu.VMEM((8, L), jnp.float32),
            pltpu.VMEM((BPW, NUM_CLASS), jnp.float32),
            pltpu.SemaphoreType.DMA,
        ],
    )
    def k(node_h, table_h, w_h, b_h, out_h, idx_v, rows_v, w_v, b_v, out_v, sem):
        wid = lax.axis_index("s") * NC + lax.axis_index("c")
        base = wid * BPW

        for j in range(N_IDX_CHUNKS):
            pltpu.sync_copy(
                node_h.at[pl.ds(base + j * IDX_CHUNK, IDX_CHUNK)], idx_v.at[j]
            )
        pltpu.sync_copy(w_h, w_v)
        pltpu.sync_copy(b_h, b_v)

        copies = [
            pltpu.async_copy(
                table_h.at[idx_v.at[j]],
                rows_v.at[pl.ds(j * IDX_CHUNK, IDX_CHUNK)],
                sem,
            )
            for j in range(N_IDX_CHUNKS)
        ]
        for c in copies:
            c.wait()

        iota = lax.iota(jnp.int32, L)

        def blk_body(blk, carry):
            rowbase = blk * ROWS_PER_BLK
            row_idx = [
                jnp.full((L,), rowbase + q * L, jnp.int32) + iota for q in range(4)
            ]

            def d_body(d, accs):
                col = jnp.full((L,), d, jnp.int32)
                gs = [plsc.load_gather(rows_v, [row_idx[q], col]) for q in range(4)]
                out = []
                for c in range(NUM_CLASS):
                    wv = w_v[c * EMB_DIM + d]
                    for q in range(4):
                        out.append(accs[c * 4 + q] + gs[q] * wv)
                return tuple(out)

            init = tuple(b_v[c] for c in range(NUM_CLASS) for _ in range(4))
            accs = lax.fori_loop(0, EMB_DIM, d_body, init)

            for c in range(NUM_CLASS):
                ccol = jnp.full((L,), c, jnp.int32)
                for q in range(4):
                    plsc.store_scatter(out_v, [row_idx[q], ccol], accs[c * 4 + q])
            return carry

        lax.fori_loop(0, N_BLKS, blk_body, 0)

        pltpu.sync_copy(out_v, out_h.at[pl.ds(base, BPW)])

    return k(node, table, w_splat, b_splat)


def kernel(node, emb_table, fc_w, fc_b):
    w_splat = jnp.broadcast_to(
        fc_w.reshape(NUM_CLASS * EMB_DIM, 1), (NUM_CLASS * EMB_DIM, L)
    )
    b_pad = jnp.concatenate([fc_b, jnp.zeros((1,), jnp.float32)])
    b_splat = jnp.broadcast_to(b_pad.reshape(8, 1), (8, L))
    return _sc_call(node, emb_table, w_splat, b_splat)
